# baseline (device time: 47138 ns/iter reference)
import jax
import jax.numpy as jnp
from jax import lax
from jax.experimental import pallas as pl
from jax.experimental.pallas import tpu as pltpu

N_DEV = 4
B = 4
SQ = 256
D = 1024
H = 8
DH = 128
T = B * SQ
HALF = T // 2
QTR = T // 4
EGT = T // 8
SCALE = 0.08838834764831843

(PH1A, PH1B, PH2A, PH2B, PH3A, PH3B,
 PH4A1, PH4A2, PH4B1, PH4B2) = range(10)


def kernel(x, Wq, Wo, Wk, Wv):
    def body(x_ref, wqkv_ref, wo_ref, out_ref,
             attn_ref, acc_ref, p16_ref, rx1_ref, tx2_ref, rx2_ref,
             fin_ref, send_sems, recv_sems):
        p = lax.axis_index("i")
        xc = p // 2
        yc = (p + xc) % 2
        py = p + 1 - 2 * (p % 2)
        px = 3 - p

        a1 = yc
        a2 = xc
        b1 = xc
        b2 = yc

        barrier_sem = pltpu.get_barrier_semaphore()
        for nbr in [py, px]:
            pl.semaphore_signal(
                barrier_sem, inc=1,
                device_id=(nbr,), device_id_type=pl.DeviceIdType.MESH,
            )
        pl.semaphore_wait(barrier_sem, 2)

        wqkv_bf = wqkv_ref[...]
        wo_bf = wo_ref[...]

        def compute_batch(batch, is_send):
            roff = batch * SQ
            xb = x_ref[pl.ds(batch, 1)][...].reshape(SQ, D)
            qkv = jnp.dot(xb, wqkv_bf,
                          preferred_element_type=jnp.float32).astype(jnp.bfloat16)
            qb = qkv[:, 0:D]
            kb = qkv[:, D:2 * D]
            vb = qkv[:, 2 * D:3 * D]
            for h in range(H):
                cols = slice(h * DH, (h + 1) * DH)
                s = lax.dot_general(
                    qb[:, cols], kb[:, cols],
                    dimension_numbers=(((1,), (1,)), ((), ())),
                    preferred_element_type=jnp.float32,
                ) * SCALE
                pexp = jnp.exp(s)
                l = jnp.sum(pexp, axis=-1, keepdims=True)
                obh = jnp.dot(pexp.astype(jnp.bfloat16), vb[:, cols],
                              preferred_element_type=jnp.float32)
                attn_ref[:, cols] = (obh / l).astype(jnp.bfloat16)
            partial_b = jnp.dot(attn_ref[...], wo_bf,
                                preferred_element_type=jnp.float32)
            if is_send:
                p16_ref[pl.ds(roff, SQ), :] = partial_b.astype(jnp.bfloat16)
            else:
                acc_ref[pl.ds(roff, SQ), :] = partial_b

        def exch(src, dst, sem_idx, target):
            return pltpu.make_async_remote_copy(
                src_ref=src, dst_ref=dst,
                send_sem=send_sems.at[sem_idx],
                recv_sem=recv_sems.at[sem_idx],
                device_id=(target,), device_id_type=pl.DeviceIdType.MESH,
            )

        compute_batch(1 - a1, True)
        compute_batch(3 - b1, True)
        r1a = exch(p16_ref.at[pl.ds((1 - a1) * QTR, QTR), :],
                   rx1_ref.at[0:QTR, :], PH1A, py)
        r1b = exch(p16_ref.at[pl.ds(HALF + (1 - b1) * QTR, QTR), :],
                   rx1_ref.at[QTR:2 * QTR, :], PH1B, px)
        r1a.start()
        r1b.start()

        ka = pl.ds(a1 * QTR, QTR)
        kb_ = pl.ds(HALF + b1 * QTR, QTR)
        oa = pl.ds(a1 * QTR + a2 * EGT, EGT)
        ob = pl.ds(HALF + b1 * QTR + b2 * EGT, EGT)
        sa2 = pl.ds(a1 * QTR + (1 - a2) * EGT, EGT)
        sb2 = pl.ds(HALF + b1 * QTR + (1 - b2) * EGT, EGT)

        compute_batch(a1, False)
        r1a.wait()
        acc_ref[ka, :] = acc_ref[ka, :] + rx1_ref[0:QTR, :].astype(jnp.float32)
        tx2_ref[0:EGT, :] = acc_ref[sa2, :].astype(jnp.bfloat16)
        r2a = exch(tx2_ref.at[0:EGT, :], rx2_ref.at[0:EGT, :], PH2A, px)
        r2a.start()
        compute_batch(2 + b1, False)
        r1b.wait()
        acc_ref[kb_, :] = acc_ref[kb_, :] + rx1_ref[QTR:2 * QTR, :].astype(jnp.float32)
        tx2_ref[EGT:2 * EGT, :] = acc_ref[sb2, :].astype(jnp.bfloat16)
        r2b = exch(tx2_ref.at[EGT:2 * EGT, :], rx2_ref.at[EGT:2 * EGT, :],
                   PH2B, py)
        r2b.start()

        r2a.wait()
        acc_ref[oa, :] = acc_ref[oa, :] + rx2_ref[0:EGT, :].astype(jnp.float32)
        fin_ref[oa, :] = acc_ref[oa, :].astype(jnp.bfloat16)
        r3a = exch(fin_ref.at[oa, :], fin_ref.at[oa, :], PH3A, px)
        r4a1 = exch(fin_ref.at[oa, :], fin_ref.at[oa, :], PH4A1, py)
        r3a.start()
        r4a1.start()
        r2b.wait()
        acc_ref[ob, :] = acc_ref[ob, :] + rx2_ref[EGT:2 * EGT, :].astype(jnp.float32)
        fin_ref[ob, :] = acc_ref[ob, :].astype(jnp.bfloat16)
        r3b = exch(fin_ref.at[ob, :], fin_ref.at[ob, :], PH3B, py)
        r4b1 = exch(fin_ref.at[ob, :], fin_ref.at[ob, :], PH4B1, px)
        r3b.start()
        r4b1.start()

        r3a.wait()
        r4a2 = exch(fin_ref.at[sa2, :], fin_ref.at[sa2, :], PH4A2, py)
        r4a2.start()
        r3b.wait()
        r4b2 = exch(fin_ref.at[sb2, :], fin_ref.at[sb2, :], PH4B2, px)
        r4b2.start()
        r4a1.wait()
        r4b1.wait()
        r4a2.wait()
        r4b2.wait()

        for b in range(B):
            out_ref[b] = fin_ref[b * SQ:(b + 1) * SQ, :]

    return pl.pallas_call(
        body,
        out_shape=jax.ShapeDtypeStruct((B, SQ, D), jnp.bfloat16),
        in_specs=[pl.BlockSpec(memory_space=pltpu.VMEM)] * 3,
        out_specs=pl.BlockSpec(memory_space=pltpu.VMEM),
        scratch_shapes=[
            pltpu.VMEM((SQ, D), jnp.bfloat16),
            pltpu.VMEM((T, D), jnp.float32),
            pltpu.VMEM((T, D), jnp.bfloat16),
            pltpu.VMEM((HALF, D), jnp.bfloat16),
            pltpu.VMEM((QTR, D), jnp.bfloat16),
            pltpu.VMEM((QTR, D), jnp.bfloat16),
            pltpu.VMEM((T, D), jnp.bfloat16),
            pltpu.SemaphoreType.DMA((10,)),
            pltpu.SemaphoreType.DMA((10,)),
        ],
        compiler_params=pltpu.CompilerParams(collective_id=0),
    )(x.astype(jnp.bfloat16),
      jnp.concatenate([Wq, Wk, Wv], axis=1).astype(jnp.bfloat16),
      Wo.astype(jnp.bfloat16))


# device time: 45702 ns/iter; 1.0314x vs baseline; 1.0314x over previous
import jax
import jax.numpy as jnp
from jax import lax
from jax.experimental import pallas as pl
from jax.experimental.pallas import tpu as pltpu

N_DEV = 4
B = 4
SQ = 256
D = 1024
H = 8
DH = 128
T = B * SQ
HALF = T // 2
QTR = T // 4
EGT = T // 8
SCALE = 0.08838834764831843

(PH1A, PH1B, PH2A, PH2B, PH3A, PH3B,
 PH4A1, PH4A2, PH4B1, PH4B2) = range(10)


def kernel(x, Wq, Wo, Wk, Wv):
    def body(x_hbm, wq_hbm, wo_hbm, wk_hbm, wv_hbm, out_ref,
             attn_ref, acc_ref, p16_ref, rx1_ref, tx2_ref, rx2_ref,
             fin_ref, xs_ref, wst_ref, xb16_ref,
             copy_sems, send_sems, recv_sems):
        p = lax.axis_index("i")
        xc = p // 2
        yc = (p + xc) % 2
        py = p + 1 - 2 * (p % 2)
        px = 3 - p

        a1 = yc
        a2 = xc
        b1 = xc
        b2 = yc

        c_x = pltpu.make_async_copy(x_hbm, xs_ref, copy_sems.at[0])
        c_q = pltpu.make_async_copy(wq_hbm, wst_ref.at[0], copy_sems.at[1])
        c_k = pltpu.make_async_copy(wk_hbm, wst_ref.at[1], copy_sems.at[2])
        c_x.start()
        c_q.start()
        c_k.start()

        barrier_sem = pltpu.get_barrier_semaphore()
        for nbr in [py, px]:
            pl.semaphore_signal(
                barrier_sem, inc=1,
                device_id=(nbr,), device_id_type=pl.DeviceIdType.MESH,
            )
        pl.semaphore_wait(barrier_sem, 2)

        c_x.wait()
        xb16_ref[...] = xs_ref[...].reshape(T, D).astype(jnp.bfloat16)
        c_q.wait()
        wq_bf = wst_ref[0].astype(jnp.bfloat16)
        c_v = pltpu.make_async_copy(wv_hbm, wst_ref.at[0], copy_sems.at[3])
        c_v.start()
        c_k.wait()
        wk_bf = wst_ref[1].astype(jnp.bfloat16)
        c_o = pltpu.make_async_copy(wo_hbm, wst_ref.at[1], copy_sems.at[4])
        c_o.start()
        c_v.wait()
        wv_bf = wst_ref[0].astype(jnp.bfloat16)
        c_o.wait()
        wo_bf = wst_ref[1].astype(jnp.bfloat16)

        def compute_batch(batch, is_send):
            roff = batch * SQ
            xb = xb16_ref[pl.ds(roff, SQ), :]
            qb = jnp.dot(xb, wq_bf,
                         preferred_element_type=jnp.float32).astype(jnp.bfloat16)
            kb = jnp.dot(xb, wk_bf,
                         preferred_element_type=jnp.float32).astype(jnp.bfloat16)
            vb = jnp.dot(xb, wv_bf,
                         preferred_element_type=jnp.float32).astype(jnp.bfloat16)
            for h in range(H):
                cols = slice(h * DH, (h + 1) * DH)
                s = lax.dot_general(
                    qb[:, cols], kb[:, cols],
                    dimension_numbers=(((1,), (1,)), ((), ())),
                    preferred_element_type=jnp.float32,
                ) * SCALE
                pexp = jnp.exp(s)
                l = jnp.sum(pexp, axis=-1, keepdims=True)
                obh = jnp.dot(pexp.astype(jnp.bfloat16), vb[:, cols],
                              preferred_element_type=jnp.float32)
                attn_ref[:, cols] = (obh / l).astype(jnp.bfloat16)
            partial_b = jnp.dot(attn_ref[...], wo_bf,
                                preferred_element_type=jnp.float32)
            if is_send:
                p16_ref[pl.ds(roff, SQ), :] = partial_b.astype(jnp.bfloat16)
            else:
                acc_ref[pl.ds(roff, SQ), :] = partial_b

        def exch(src, dst, sem_idx, target):
            return pltpu.make_async_remote_copy(
                src_ref=src, dst_ref=dst,
                send_sem=send_sems.at[sem_idx],
                recv_sem=recv_sems.at[sem_idx],
                device_id=(target,), device_id_type=pl.DeviceIdType.MESH,
            )

        compute_batch(1 - a1, True)
        compute_batch(3 - b1, True)
        r1a = exch(p16_ref.at[pl.ds((1 - a1) * QTR, QTR), :],
                   rx1_ref.at[0:QTR, :], PH1A, py)
        r1b = exch(p16_ref.at[pl.ds(HALF + (1 - b1) * QTR, QTR), :],
                   rx1_ref.at[QTR:2 * QTR, :], PH1B, px)
        r1a.start()
        r1b.start()

        ka = pl.ds(a1 * QTR, QTR)
        kb_ = pl.ds(HALF + b1 * QTR, QTR)
        oa = pl.ds(a1 * QTR + a2 * EGT, EGT)
        ob = pl.ds(HALF + b1 * QTR + b2 * EGT, EGT)
        sa2 = pl.ds(a1 * QTR + (1 - a2) * EGT, EGT)
        sb2 = pl.ds(HALF + b1 * QTR + (1 - b2) * EGT, EGT)

        compute_batch(a1, False)
        r1a.wait()
        acc_ref[ka, :] = acc_ref[ka, :] + rx1_ref[0:QTR, :].astype(jnp.float32)
        tx2_ref[0:EGT, :] = acc_ref[sa2, :].astype(jnp.bfloat16)
        r2a = exch(tx2_ref.at[0:EGT, :], rx2_ref.at[0:EGT, :], PH2A, px)
        r2a.start()
        compute_batch(2 + b1, False)
        r1b.wait()
        acc_ref[kb_, :] = acc_ref[kb_, :] + rx1_ref[QTR:2 * QTR, :].astype(jnp.float32)
        tx2_ref[EGT:2 * EGT, :] = acc_ref[sb2, :].astype(jnp.bfloat16)
        r2b = exch(tx2_ref.at[EGT:2 * EGT, :], rx2_ref.at[EGT:2 * EGT, :],
                   PH2B, py)
        r2b.start()

        r2a.wait()
        acc_ref[oa, :] = acc_ref[oa, :] + rx2_ref[0:EGT, :].astype(jnp.float32)
        fin_ref[oa, :] = acc_ref[oa, :].astype(jnp.bfloat16)
        r3a = exch(fin_ref.at[oa, :], fin_ref.at[oa, :], PH3A, px)
        r4a1 = exch(fin_ref.at[oa, :], fin_ref.at[oa, :], PH4A1, py)
        r3a.start()
        r4a1.start()
        r2b.wait()
        acc_ref[ob, :] = acc_ref[ob, :] + rx2_ref[EGT:2 * EGT, :].astype(jnp.float32)
        fin_ref[ob, :] = acc_ref[ob, :].astype(jnp.bfloat16)
        r3b = exch(fin_ref.at[ob, :], fin_ref.at[ob, :], PH3B, py)
        r4b1 = exch(fin_ref.at[ob, :], fin_ref.at[ob, :], PH4B1, px)
        r3b.start()
        r4b1.start()

        r3a.wait()
        r4a2 = exch(fin_ref.at[sa2, :], fin_ref.at[sa2, :], PH4A2, py)
        r4a2.start()
        r3b.wait()
        r4b2 = exch(fin_ref.at[sb2, :], fin_ref.at[sb2, :], PH4B2, px)
        r4b2.start()
        r4a1.wait()
        r4b1.wait()
        r4a2.wait()
        r4b2.wait()

        for b in range(B):
            out_ref[b] = fin_ref[b * SQ:(b + 1) * SQ, :]

    return pl.pallas_call(
        body,
        out_shape=jax.ShapeDtypeStruct((B, SQ, D), jnp.bfloat16),
        in_specs=[pl.BlockSpec(memory_space=pltpu.MemorySpace.HBM)] * 5,
        out_specs=pl.BlockSpec(memory_space=pltpu.VMEM),
        scratch_shapes=[
            pltpu.VMEM((SQ, D), jnp.bfloat16),
            pltpu.VMEM((T, D), jnp.float32),
            pltpu.VMEM((T, D), jnp.bfloat16),
            pltpu.VMEM((HALF, D), jnp.bfloat16),
            pltpu.VMEM((QTR, D), jnp.bfloat16),
            pltpu.VMEM((QTR, D), jnp.bfloat16),
            pltpu.VMEM((T, D), jnp.bfloat16),
            pltpu.VMEM((B, SQ, D), jnp.float32),
            pltpu.VMEM((2, D, D), jnp.float32),
            pltpu.VMEM((T, D), jnp.bfloat16),
            pltpu.SemaphoreType.DMA((5,)),
            pltpu.SemaphoreType.DMA((10,)),
            pltpu.SemaphoreType.DMA((10,)),
        ],
        compiler_params=pltpu.CompilerParams(
            collective_id=0, vmem_limit_bytes=56 * 1024 * 1024),
    )(x, Wq, Wo, Wk, Wv)


# device time: 45015 ns/iter; 1.0472x vs baseline; 1.0153x over previous
import jax
import jax.numpy as jnp
from jax import lax
from jax.experimental import pallas as pl
from jax.experimental.pallas import tpu as pltpu

N_DEV = 4
B = 4
SQ = 256
D = 1024
H = 8
DH = 128
T = B * SQ
HALF = T // 2
QTR = T // 4
EGT = T // 8
SCALE = 0.08838834764831843

(PH1A, PH1B, PH2A, PH2B, PH3A, PH3B,
 PH4A1, PH4A2, PH4B1, PH4B2) = range(10)


def kernel(x, Wq, Wo, Wk, Wv):
    def body(x_hbm, wq_hbm, wo_hbm, wk_hbm, wv_hbm, out_ref,
             attn_ref, acc_ref, p16_ref, rx1_ref, tx2_ref, rx2_ref,
             fin_ref, xs_ref, wst_ref, xb16_ref,
             copy_sems, send_sems, recv_sems):
        p = lax.axis_index("i")
        xc = p // 2
        yc = (p + xc) % 2
        py = p + 1 - 2 * (p % 2)
        px = 3 - p

        a1 = yc
        a2 = xc
        b1 = xc
        b2 = yc

        c_x = pltpu.make_async_copy(x_hbm, xs_ref, copy_sems.at[0])
        c_q = pltpu.make_async_copy(wq_hbm, wst_ref.at[0], copy_sems.at[1])
        c_k = pltpu.make_async_copy(wk_hbm, wst_ref.at[1], copy_sems.at[2])
        c_x.start()
        c_q.start()
        c_k.start()

        barrier_sem = pltpu.get_barrier_semaphore()
        for nbr in [py, px]:
            pl.semaphore_signal(
                barrier_sem, inc=1,
                device_id=(nbr,), device_id_type=pl.DeviceIdType.MESH,
            )
        pl.semaphore_wait(barrier_sem, 2)

        c_x.wait()
        xb16_ref[...] = xs_ref[...].reshape(T, D).astype(jnp.bfloat16)
        c_q.wait()
        wq_bf = wst_ref[0].astype(jnp.bfloat16)
        c_v = pltpu.make_async_copy(wv_hbm, wst_ref.at[0], copy_sems.at[3])
        c_v.start()
        c_k.wait()
        wk_bf = wst_ref[1].astype(jnp.bfloat16)
        c_o = pltpu.make_async_copy(wo_hbm, wst_ref.at[1], copy_sems.at[4])
        c_o.start()
        c_v.wait()
        wv_bf = wst_ref[0].astype(jnp.bfloat16)
        c_o.wait()
        wo_bf = wst_ref[1].astype(jnp.bfloat16)

        def compute_rows(q_roff, n_rows, batch_roff):
            xb = xb16_ref[pl.ds(batch_roff, SQ), :]
            xq = xb16_ref[pl.ds(q_roff, n_rows), :]
            qb = jnp.dot(xq, wq_bf,
                         preferred_element_type=jnp.float32).astype(jnp.bfloat16)
            kb = jnp.dot(xb, wk_bf,
                         preferred_element_type=jnp.float32).astype(jnp.bfloat16)
            vb = jnp.dot(xb, wv_bf,
                         preferred_element_type=jnp.float32).astype(jnp.bfloat16)
            for h in range(H):
                cols = slice(h * DH, (h + 1) * DH)
                s = lax.dot_general(
                    qb[:, cols], kb[:, cols],
                    dimension_numbers=(((1,), (1,)), ((), ())),
                    preferred_element_type=jnp.float32,
                ) * SCALE
                pexp = jnp.exp(s)
                l = jnp.sum(pexp, axis=-1, keepdims=True)
                obh = jnp.dot(pexp.astype(jnp.bfloat16), vb[:, cols],
                              preferred_element_type=jnp.float32)
                attn_ref[0:n_rows, cols] = (obh / l).astype(jnp.bfloat16)
            return jnp.dot(attn_ref[0:n_rows, :], wo_bf,
                           preferred_element_type=jnp.float32)

        def compute_batch(batch, is_send):
            roff = batch * SQ
            partial_b = compute_rows(roff, SQ, roff)
            if is_send:
                p16_ref[pl.ds(roff, SQ), :] = partial_b.astype(jnp.bfloat16)
            else:
                acc_ref[pl.ds(roff, SQ), :] = partial_b

        def exch(src, dst, sem_idx, target):
            return pltpu.make_async_remote_copy(
                src_ref=src, dst_ref=dst,
                send_sem=send_sems.at[sem_idx],
                recv_sem=recv_sems.at[sem_idx],
                device_id=(target,), device_id_type=pl.DeviceIdType.MESH,
            )

        compute_batch(1 - a1, True)
        compute_batch(3 - b1, True)
        r1a = exch(p16_ref.at[pl.ds((1 - a1) * QTR, QTR), :],
                   rx1_ref.at[0:QTR, :], PH1A, py)
        r1b = exch(p16_ref.at[pl.ds(HALF + (1 - b1) * QTR, QTR), :],
                   rx1_ref.at[QTR:2 * QTR, :], PH1B, px)
        r1a.start()
        r1b.start()

        ka = pl.ds(a1 * QTR, QTR)
        kb_ = pl.ds(HALF + b1 * QTR, QTR)
        oa = pl.ds(a1 * QTR + a2 * EGT, EGT)
        ob = pl.ds(HALF + b1 * QTR + b2 * EGT, EGT)
        sa2 = pl.ds(a1 * QTR + (1 - a2) * EGT, EGT)
        sb2 = pl.ds(HALF + b1 * QTR + (1 - b2) * EGT, EGT)

        compute_batch(a1, False)
        r1a.wait()
        acc_ref[ka, :] = acc_ref[ka, :] + rx1_ref[0:QTR, :].astype(jnp.float32)
        tx2_ref[0:EGT, :] = acc_ref[sa2, :].astype(jnp.bfloat16)
        r2a = exch(tx2_ref.at[0:EGT, :], rx2_ref.at[0:EGT, :], PH2A, px)
        r2a.start()
        broff = HALF + b1 * QTR
        sb2_start = broff + (1 - b2) * EGT
        ob_start = broff + b2 * EGT
        p1 = compute_rows(sb2_start, EGT, broff)
        r1b.wait()
        tx2_ref[EGT:2 * EGT, :] = (
            p1 + rx1_ref[pl.ds(QTR + (1 - b2) * EGT, EGT), :].astype(jnp.float32)
        ).astype(jnp.bfloat16)
        r2b = exch(tx2_ref.at[EGT:2 * EGT, :], rx2_ref.at[EGT:2 * EGT, :],
                   PH2B, py)
        r2b.start()
        p2 = compute_rows(ob_start, EGT, broff)
        acc_ref[ob, :] = p2 + rx1_ref[pl.ds(QTR + b2 * EGT, EGT), :].astype(jnp.float32)

        r2a.wait()
        acc_ref[oa, :] = acc_ref[oa, :] + rx2_ref[0:EGT, :].astype(jnp.float32)
        fin_ref[oa, :] = acc_ref[oa, :].astype(jnp.bfloat16)
        r3a = exch(fin_ref.at[oa, :], fin_ref.at[oa, :], PH3A, px)
        r4a1 = exch(fin_ref.at[oa, :], fin_ref.at[oa, :], PH4A1, py)
        r3a.start()
        r4a1.start()
        r2b.wait()
        acc_ref[ob, :] = acc_ref[ob, :] + rx2_ref[EGT:2 * EGT, :].astype(jnp.float32)
        fin_ref[ob, :] = acc_ref[ob, :].astype(jnp.bfloat16)
        r3b = exch(fin_ref.at[ob, :], fin_ref.at[ob, :], PH3B, py)
        r4b1 = exch(fin_ref.at[ob, :], fin_ref.at[ob, :], PH4B1, px)
        r3b.start()
        r4b1.start()

        r3a.wait()
        r4a2 = exch(fin_ref.at[sa2, :], fin_ref.at[sa2, :], PH4A2, py)
        r4a2.start()
        r3b.wait()
        r4b2 = exch(fin_ref.at[sb2, :], fin_ref.at[sb2, :], PH4B2, px)
        r4b2.start()
        r4a1.wait()
        r4b1.wait()
        r4a2.wait()
        r4b2.wait()

        for b in range(B):
            out_ref[b] = fin_ref[b * SQ:(b + 1) * SQ, :]

    return pl.pallas_call(
        body,
        out_shape=jax.ShapeDtypeStruct((B, SQ, D), jnp.bfloat16),
        in_specs=[pl.BlockSpec(memory_space=pltpu.MemorySpace.HBM)] * 5,
        out_specs=pl.BlockSpec(memory_space=pltpu.VMEM),
        scratch_shapes=[
            pltpu.VMEM((SQ, D), jnp.bfloat16),
            pltpu.VMEM((T, D), jnp.float32),
            pltpu.VMEM((T, D), jnp.bfloat16),
            pltpu.VMEM((HALF, D), jnp.bfloat16),
            pltpu.VMEM((QTR, D), jnp.bfloat16),
            pltpu.VMEM((QTR, D), jnp.bfloat16),
            pltpu.VMEM((T, D), jnp.bfloat16),
            pltpu.VMEM((B, SQ, D), jnp.float32),
            pltpu.VMEM((2, D, D), jnp.float32),
            pltpu.VMEM((T, D), jnp.bfloat16),
            pltpu.SemaphoreType.DMA((5,)),
            pltpu.SemaphoreType.DMA((10,)),
            pltpu.SemaphoreType.DMA((10,)),
        ],
        compiler_params=pltpu.CompilerParams(
            collective_id=0, vmem_limit_bytes=56 * 1024 * 1024),
    )(x, Wq, Wo, Wk, Wv)


# device time: 44941 ns/iter; 1.0489x vs baseline; 1.0016x over previous
import jax
import jax.numpy as jnp
from jax import lax
from jax.experimental import pallas as pl
from jax.experimental.pallas import tpu as pltpu

N_DEV = 4
B = 4
SQ = 256
D = 1024
H = 8
DH = 128
T = B * SQ
HALF = T // 2
QTR = T // 4
EGT = T // 8
SCALE = 0.08838834764831843

(PH1A, PH1B, PH2A, PH2B, PH3A, PH3B,
 PH4A1, PH4A2, PH4B1, PH4B2) = range(10)


def kernel(x, Wq, Wo, Wk, Wv):
    def body(x_hbm, wq_hbm, wo_hbm, wk_hbm, wv_hbm, out_ref,
             attn_ref, acc_ref, p16_ref, rx1_ref, tx2_ref, rx2_ref,
             fin_ref, xs_ref, wst_ref, xb16_ref,
             copy_sems, send_sems, recv_sems):
        p = lax.axis_index("i")
        xc = p // 2
        yc = (p + xc) % 2
        py = p + 1 - 2 * (p % 2)
        px = 3 - p

        a1 = yc
        a2 = xc
        b1 = xc
        b2 = yc

        c_x = pltpu.make_async_copy(x_hbm, xs_ref, copy_sems.at[0])
        c_q = pltpu.make_async_copy(wq_hbm, wst_ref.at[0], copy_sems.at[1])
        c_k = pltpu.make_async_copy(wk_hbm, wst_ref.at[1], copy_sems.at[2])
        c_x.start()
        c_q.start()
        c_k.start()

        barrier_sem = pltpu.get_barrier_semaphore()
        for nbr in [py, px]:
            pl.semaphore_signal(
                barrier_sem, inc=1,
                device_id=(nbr,), device_id_type=pl.DeviceIdType.MESH,
            )
        pl.semaphore_wait(barrier_sem, 2)

        c_x.wait()
        xb16_ref[...] = xs_ref[...].reshape(T, D).astype(jnp.bfloat16)
        c_q.wait()
        wq_bf = wst_ref[0].astype(jnp.bfloat16)
        c_v = pltpu.make_async_copy(wv_hbm, wst_ref.at[0], copy_sems.at[3])
        c_v.start()
        c_k.wait()
        wk_bf = wst_ref[1].astype(jnp.bfloat16)
        c_o = pltpu.make_async_copy(wo_hbm, wst_ref.at[1], copy_sems.at[4])
        c_o.start()
        c_v.wait()
        wv_bf = wst_ref[0].astype(jnp.bfloat16)
        c_o.wait()
        wo_bf = wst_ref[1].astype(jnp.bfloat16)

        def compute_rows(q_roff, n_rows, batch_roff):
            xb = xb16_ref[pl.ds(batch_roff, SQ), :]
            xq = xb16_ref[pl.ds(q_roff, n_rows), :]
            qb = jnp.dot(xq, wq_bf,
                         preferred_element_type=jnp.float32).astype(jnp.bfloat16)
            kb = jnp.dot(xb, wk_bf,
                         preferred_element_type=jnp.float32).astype(jnp.bfloat16)
            vb = jnp.dot(xb, wv_bf,
                         preferred_element_type=jnp.float32).astype(jnp.bfloat16)
            for h in range(H):
                cols = slice(h * DH, (h + 1) * DH)
                s = lax.dot_general(
                    qb[:, cols], kb[:, cols],
                    dimension_numbers=(((1,), (1,)), ((), ())),
                    preferred_element_type=jnp.float32,
                ) * SCALE
                pexp = jnp.exp(s)
                l = jnp.sum(pexp, axis=-1, keepdims=True)
                obh = jnp.dot(pexp.astype(jnp.bfloat16), vb[:, cols],
                              preferred_element_type=jnp.float32)
                attn_ref[0:n_rows, cols] = (obh / l).astype(jnp.bfloat16)
            return jnp.dot(attn_ref[0:n_rows, :], wo_bf,
                           preferred_element_type=jnp.float32)

        def compute_batch(batch, is_send):
            roff = batch * SQ
            partial_b = compute_rows(roff, SQ, roff)
            if is_send:
                p16_ref[pl.ds(roff, SQ), :] = partial_b.astype(jnp.bfloat16)
            else:
                acc_ref[pl.ds(roff, SQ), :] = partial_b

        def exch(src, dst, sem_idx, target):
            return pltpu.make_async_remote_copy(
                src_ref=src, dst_ref=dst,
                send_sem=send_sems.at[sem_idx],
                recv_sem=recv_sems.at[sem_idx],
                device_id=(target,), device_id_type=pl.DeviceIdType.MESH,
            )

        compute_batch(1 - a1, True)
        compute_batch(3 - b1, True)
        r1a = exch(p16_ref.at[pl.ds((1 - a1) * QTR, QTR), :],
                   rx1_ref.at[0:QTR, :], PH1A, py)
        r1b = exch(p16_ref.at[pl.ds(HALF + (1 - b1) * QTR, QTR), :],
                   rx1_ref.at[QTR:2 * QTR, :], PH1B, px)
        r1a.start()
        r1b.start()

        ka = pl.ds(a1 * QTR, QTR)
        oa = pl.ds(a1 * QTR + a2 * EGT, EGT)
        ob = pl.ds(HALF + b1 * QTR + b2 * EGT, EGT)
        sa2 = pl.ds(a1 * QTR + (1 - a2) * EGT, EGT)
        sb2 = pl.ds(HALF + b1 * QTR + (1 - b2) * EGT, EGT)

        compute_batch(a1, False)
        r1a.wait()
        acc_ref[ka, :] = acc_ref[ka, :] + rx1_ref[0:QTR, :].astype(jnp.float32)
        tx2_ref[0:EGT, :] = acc_ref[sa2, :].astype(jnp.bfloat16)
        r2a = exch(tx2_ref.at[0:EGT, :], rx2_ref.at[0:EGT, :], PH2A, px)
        r2a.start()
        broff = HALF + b1 * QTR
        sb2_start = broff + (1 - b2) * EGT
        ob_start = broff + b2 * EGT
        p1 = compute_rows(sb2_start, EGT, broff)
        r1b.wait()
        tx2_ref[EGT:2 * EGT, :] = (
            p1 + rx1_ref[pl.ds(QTR + (1 - b2) * EGT, EGT), :].astype(jnp.float32)
        ).astype(jnp.bfloat16)
        r2b = exch(tx2_ref.at[EGT:2 * EGT, :], rx2_ref.at[EGT:2 * EGT, :],
                   PH2B, py)
        r2b.start()
        p2 = compute_rows(ob_start, EGT, broff)
        acc_ref[ob, :] = p2 + rx1_ref[pl.ds(QTR + b2 * EGT, EGT), :].astype(jnp.float32)

        r2a.wait()
        acc_ref[oa, :] = acc_ref[oa, :] + rx2_ref[0:EGT, :].astype(jnp.float32)
        fin_ref[oa, :] = acc_ref[oa, :].astype(jnp.bfloat16)
        r3a = exch(fin_ref.at[oa, :], fin_ref.at[oa, :], PH3A, px)
        r4a1 = exch(fin_ref.at[oa, :], fin_ref.at[oa, :], PH4A1, py)
        r3a.start()
        r4a1.start()
        r2b.wait()
        acc_ref[ob, :] = acc_ref[ob, :] + rx2_ref[EGT:2 * EGT, :].astype(jnp.float32)
        fin_ref[ob, :] = acc_ref[ob, :].astype(jnp.bfloat16)
        r3b = exch(fin_ref.at[ob, :], fin_ref.at[ob, :], PH3B, py)
        r4b1 = exch(fin_ref.at[ob, :], fin_ref.at[ob, :], PH4B1, px)
        r3b.start()
        r4b1.start()

        r3a.wait()
        r4a2 = exch(fin_ref.at[sa2, :], fin_ref.at[sa2, :], PH4A2, py)
        r4a2.start()
        r3b.wait()
        r4b2 = exch(fin_ref.at[sb2, :], fin_ref.at[sb2, :], PH4B2, px)
        r4b2.start()
        r4a1.wait()
        r4b1.wait()
        r4a2.wait()
        r4b2.wait()

        for b in range(B):
            out_ref[b] = fin_ref[b * SQ:(b + 1) * SQ, :]

    return pl.pallas_call(
        body,
        out_shape=jax.ShapeDtypeStruct((B, SQ, D), jnp.bfloat16),
        in_specs=[pl.BlockSpec(memory_space=pltpu.MemorySpace.HBM)] * 5,
        out_specs=pl.BlockSpec(memory_space=pltpu.VMEM),
        scratch_shapes=[
            pltpu.VMEM((SQ, D), jnp.bfloat16),
            pltpu.VMEM((T, D), jnp.float32),
            pltpu.VMEM((T, D), jnp.bfloat16),
            pltpu.VMEM((HALF, D), jnp.bfloat16),
            pltpu.VMEM((QTR, D), jnp.bfloat16),
            pltpu.VMEM((QTR, D), jnp.bfloat16),
            pltpu.VMEM((T, D), jnp.bfloat16),
            pltpu.VMEM((B, SQ, D), jnp.float32),
            pltpu.VMEM((2, D, D), jnp.float32),
            pltpu.VMEM((T, D), jnp.bfloat16),
            pltpu.SemaphoreType.DMA((5,)),
            pltpu.SemaphoreType.DMA((10,)),
            pltpu.SemaphoreType.DMA((10,)),
        ],
        compiler_params=pltpu.CompilerParams(
            collective_id=0, vmem_limit_bytes=56 * 1024 * 1024),
    )(x, Wq, Wo, Wk, Wv)
